# baseline (speedup 1.0000x reference)
import jax
import jax.numpy as jnp
from jax import lax
from jax.experimental import pallas as pl
from jax.experimental.pallas import tpu as pltpu
from jax.experimental.pallas import tpu_sc as plsc

N = 10000
D = 128
H = 64
HEAD = 128
E = 320000

NC = 2
NS = 16
NW = NC * NS
CH = 125
NCH = 80
EPW = CH * NCH
NPAD = 10240
RPW = NPAD // NS
CPR = RPW // 128

_f32 = jnp.float32


def _mm(a, b):
  return jnp.dot(a, b, preferred_element_type=_f32)


def _sc_scatter_kernel(gather: bool, width: int):
  mesh = plsc.VectorSubcoreMesh(core_axis_name="c", subcore_axis_name="s")
  w16 = width // 16

  def body(*refs):
    if gather:
      y_hbm, rows_hbm, cols_hbm, out_hbm, rows_v, cols_v, gbuf, zbuf, s_sh, \
          gsem0, gsem1, ssem0, ssem1 = refs
    else:
      cols_hbm, out_hbm, cols_v, gbuf, zbuf, s_sh, gsem0, gsem1, ssem0, ssem1 = refs
    cid = lax.axis_index("c")
    sid = lax.axis_index("s")
    wid = cid * NS + sid

    if gather:
      pltpu.sync_copy(rows_hbm.at[wid], rows_v)
    pltpu.sync_copy(cols_hbm.at[wid], cols_v)

    if not gather:
      def _fill_row(r):
        for c in range(w16):
          gbuf[0, r, pl.ds(c * 16, 16)] = jnp.ones((16,), _f32)
          gbuf[1, r, pl.ds(c * 16, 16)] = jnp.ones((16,), _f32)

      pl.loop(0, CH)(_fill_row)

    def _zero_row(r):
      for c in range(w16):
        zbuf[r, pl.ds(c * 16, 16)] = jnp.zeros((16,), _f32)

    pl.loop(0, 128)(_zero_row)
    for t in range(CPR):
      pltpu.sync_copy(zbuf, s_sh.at[pl.ds(sid * RPW + t * 128, 128)])
    plsc.subcore_barrier()

    if gather:
      pltpu.async_copy(y_hbm.at[rows_v.at[0]], gbuf.at[0], gsem0)
      pltpu.async_copy(y_hbm.at[rows_v.at[1]], gbuf.at[1], gsem1)

      def _step(j):
        pltpu.make_async_copy(y_hbm.at[rows_v.at[j]], gbuf.at[0], gsem0).wait()
        pltpu.async_copy(gbuf.at[0], s_sh.at[cols_v.at[j]], ssem0, add=True)
        pltpu.make_async_copy(y_hbm.at[rows_v.at[j + 1]], gbuf.at[1], gsem1).wait()
        pltpu.async_copy(gbuf.at[1], s_sh.at[cols_v.at[j + 1]], ssem1, add=True)
        pltpu.make_async_copy(gbuf.at[0], s_sh.at[cols_v.at[j]], ssem0).wait()

        @pl.when(j + 2 < NCH)
        def _():
          pltpu.async_copy(y_hbm.at[rows_v.at[j + 2]], gbuf.at[0], gsem0)

        pltpu.make_async_copy(gbuf.at[1], s_sh.at[cols_v.at[j + 1]], ssem1).wait()

        @pl.when(j + 3 < NCH)
        def _():
          pltpu.async_copy(y_hbm.at[rows_v.at[j + 3]], gbuf.at[1], gsem1)

      pl.loop(0, NCH, step=2)(_step)
    else:
      def _step(j):
        pltpu.async_copy(gbuf.at[0], s_sh.at[cols_v.at[j]], ssem0, add=True)
        pltpu.async_copy(gbuf.at[1], s_sh.at[cols_v.at[j + 1]], ssem1, add=True)
        pltpu.make_async_copy(gbuf.at[0], s_sh.at[cols_v.at[j]], ssem0).wait()
        pltpu.make_async_copy(gbuf.at[1], s_sh.at[cols_v.at[j + 1]], ssem1).wait()

      pl.loop(0, NCH, step=2)(_step)

    plsc.subcore_barrier()
    for t in range(CPR):
      base = sid * RPW + t * 128
      pltpu.sync_copy(s_sh.at[pl.ds(base, 128)], zbuf)
      pltpu.sync_copy(zbuf, out_hbm.at[cid, pl.ds(base, 128)])

  scratch = []
  if gather:
    scratch.append(pltpu.VMEM((NCH, CH), jnp.int32))
  scratch += [
      pltpu.VMEM((NCH, CH), jnp.int32),
      pltpu.VMEM((2, CH, width), _f32),
      pltpu.VMEM((128, width), _f32),
      pltpu.VMEM_SHARED((NPAD, width), _f32),
      pltpu.SemaphoreType.DMA,
      pltpu.SemaphoreType.DMA,
      pltpu.SemaphoreType.DMA,
      pltpu.SemaphoreType.DMA,
  ]
  return pl.kernel(
      body,
      out_type=jax.ShapeDtypeStruct((NC, NPAD, width), _f32),
      mesh=mesh,
      scratch_types=scratch,
      compiler_params=pltpu.CompilerParams(use_tc_tiling_on_sc=False),
  )


def _tc_mm1_body(x_ref, w1_ref, xw_ref):
  xw_ref[...] = _mm(x_ref[...], w1_ref[...])


def _tc_scale1_body(sdeg_ref, xw_ref, y_ref, dinv_ref):
  deg = sdeg_ref[0, :, 0:1] + sdeg_ref[1, :, 0:1]
  dinv = lax.rsqrt(deg + 1.0)
  y_ref[...] = dinv * xw_ref[...]
  dinv_ref[...] = dinv


def _tc_mid_body(s_ref, y_ref, dinv_ref, b_ref, w_ref, out_ref):
  s = s_ref[0] + s_ref[1] + y_ref[...]
  dinv = dinv_ref[...]
  h = jax.nn.relu(dinv * s + b_ref[...])
  out_ref[...] = dinv * _mm(h, w_ref[...])


def _tc_final_body(s_ref, y_ref, dinv_ref, b_ref, wf1_ref, bf1_ref,
                   wf2_ref, bf2_ref, out_ref):
  s = s_ref[0] + s_ref[1] + y_ref[...]
  h = jax.nn.relu(dinv_ref[...] * s + b_ref[...])
  mask = lax.broadcasted_iota(jnp.int32, (NPAD, 1), 0) < N
  h = jnp.where(mask, h, 0.0)
  pooled = jnp.sum(h, axis=0, keepdims=True) * (1.0 / N)
  hidden = jax.nn.relu(_mm(pooled, wf1_ref[...]) + bf1_ref[...])
  out_ref[...] = _mm(hidden, wf2_ref[...]) + bf2_ref[...]


_BLK = 1024
_GRID = NPAD // _BLK


def _tc_mm1(x_pad, w1t):
  return pl.pallas_call(
      _tc_mm1_body,
      grid=(_GRID,),
      in_specs=[
          pl.BlockSpec((_BLK, D), lambda i: (i, 0)),
          pl.BlockSpec((D, H), lambda i: (0, 0)),
      ],
      out_specs=pl.BlockSpec((_BLK, H), lambda i: (i, 0)),
      out_shape=jax.ShapeDtypeStruct((NPAD, H), _f32),
  )(x_pad, w1t)


def _tc_scale1(sdeg, xw):
  return pl.pallas_call(
      _tc_scale1_body,
      grid=(_GRID,),
      in_specs=[
          pl.BlockSpec((NC, _BLK, 16), lambda i: (0, i, 0)),
          pl.BlockSpec((_BLK, H), lambda i: (i, 0)),
      ],
      out_specs=[
          pl.BlockSpec((_BLK, H), lambda i: (i, 0)),
          pl.BlockSpec((_BLK, 1), lambda i: (i, 0)),
      ],
      out_shape=[
          jax.ShapeDtypeStruct((NPAD, H), _f32),
          jax.ShapeDtypeStruct((NPAD, 1), _f32),
      ],
  )(sdeg, xw)


def _tc_mid(s, y, dinv, b2d, w):
  return pl.pallas_call(
      _tc_mid_body,
      grid=(_GRID,),
      in_specs=[
          pl.BlockSpec((NC, _BLK, H), lambda i: (0, i, 0)),
          pl.BlockSpec((_BLK, H), lambda i: (i, 0)),
          pl.BlockSpec((_BLK, 1), lambda i: (i, 0)),
          pl.BlockSpec((1, H), lambda i: (0, 0)),
          pl.BlockSpec((H, H), lambda i: (0, 0)),
      ],
      out_specs=pl.BlockSpec((_BLK, H), lambda i: (i, 0)),
      out_shape=jax.ShapeDtypeStruct((NPAD, H), _f32),
  )(s, y, dinv, b2d, w)


def _tc_final(s, y, dinv, b2d, Wf1, bf1_2d, Wf2, bf2_2d):
  return pl.pallas_call(
      _tc_final_body,
      out_shape=jax.ShapeDtypeStruct((1, 1), _f32),
  )(s, y, dinv, b2d, Wf1, bf1_2d, Wf2, bf2_2d)


@jax.jit
def _run(graph, edge_index, W1, b1, W2, b2, W3, b3, Wf1, bf1, Wf2, bf2):
  x_pad = jnp.pad(graph, ((0, NPAD - N), (0, 0)))
  rows3d = edge_index[0].reshape(NW, NCH, CH)
  cols3d = edge_index[1].reshape(NW, NCH, CH)

  b1_2d = b1.reshape(1, H)
  b2_2d = b2.reshape(1, H)
  b3_2d = b3.reshape(1, H)
  bf1_2d = bf1.reshape(1, HEAD)
  bf2_2d = bf2.reshape(1, 1)

  w1t = W1.T
  w2t = W2.T
  w3t = W3.T
  wf1t = Wf1.T
  wf2t = Wf2.T

  deg_pass = _sc_scatter_kernel(gather=False, width=16)
  agg_pass = _sc_scatter_kernel(gather=True, width=H)

  sdeg = deg_pass(cols3d)
  xw1 = _tc_mm1(x_pad, w1t)
  y1, dinv = _tc_scale1(sdeg, xw1)
  s1 = agg_pass(y1, rows3d, cols3d)
  y2 = _tc_mid(s1, y1, dinv, b1_2d, w2t)
  s2 = agg_pass(y2, rows3d, cols3d)
  y3 = _tc_mid(s2, y2, dinv, b2_2d, w3t)
  s3 = agg_pass(y3, rows3d, cols3d)
  return _tc_final(s3, y3, dinv, b3_2d, wf1t, bf1_2d, wf2t, bf2_2d)


def kernel(graph, edge_index, batch, W1, b1, W2, b2, W3, b3, Wf1, bf1, Wf2, bf2):
  del batch
  return _run(graph, edge_index, W1, b1, W2, b2, W3, b3, Wf1, bf1, Wf2, bf2)
